# Initial kernel scaffold; baseline (speedup 1.0000x reference)
import jax
import jax.numpy as jnp
from jax.experimental import pallas as pl


def kernel(inputs, Wg, bg, We, be):
    raise NotImplementedError("write your pallas kernel here")


# speedup vs baseline: 1.8802x; 1.8802x over previous
import functools

import jax
import jax.numpy as jnp
from jax import lax
from jax.experimental import pallas as pl
from jax.experimental.pallas import tpu as pltpu
from jax.experimental.pallas import tpu_sc as plsc

_K = 2
_E = 8
_D = 1024
_T = 4096
_B = 256
_P = _T * _K + _E * _B
_NB = _P // _B
_TT = 512

_NW = 32
_TPW = _T // _NW
_CH = 32
_NCH = _TPW // _CH

_mesh = plsc.VectorSubcoreMesh(core_axis_name="c", subcore_axis_name="s")


def _router_body(x_ref, wg_ref, bg_ref, idx_ref, prob_ref):
    scores = jnp.dot(x_ref[...], wg_ref[...],
                     preferred_element_type=jnp.float32) + bg_ref[...]
    col = lax.broadcasted_iota(jnp.int32, scores.shape, 1)
    s1 = jnp.max(scores, axis=1, keepdims=True)
    i1 = jnp.min(jnp.where(scores == s1, col, _E), axis=1, keepdims=True)
    masked = jnp.where(col == i1, -jnp.inf, scores)
    s2 = jnp.max(masked, axis=1, keepdims=True)
    i2 = jnp.min(jnp.where(masked == s2, col, _E), axis=1, keepdims=True)
    e2 = jnp.exp(s2 - s1)
    denom = 1.0 + e2
    idx_ref[:, 0:1] = i1
    idx_ref[:, 1:2] = i2
    prob_ref[:, 0:1] = 1.0 / denom
    prob_ref[:, 1:2] = e2 / denom


def _router(x, wg, bg2):
    return pl.pallas_call(
        _router_body,
        grid=(_T // _TT,),
        in_specs=[
            pl.BlockSpec((_TT, _D), lambda t: (t, 0)),
            pl.BlockSpec((_D, _E), lambda t: (0, 0)),
            pl.BlockSpec((1, _E), lambda t: (0, 0)),
        ],
        out_specs=[
            pl.BlockSpec((_TT, _K), lambda t: (t, 0)),
            pl.BlockSpec((_TT, _K), lambda t: (t, 0)),
        ],
        out_shape=[
            jax.ShapeDtypeStruct((_T, _K), jnp.int32),
            jax.ShapeDtypeStruct((_T, _K), jnp.float32),
        ],
    )(x, wg, bg2)


def _dispatch_plan(idx):
    flat_e = idx.reshape(-1)
    onehot = (flat_e[:, None] == jnp.arange(_E)[None, :]).astype(jnp.int32)
    counts = jnp.sum(onehot, axis=0)
    padded = ((counts + _B - 1) // _B) * _B
    starts = jnp.concatenate(
        [jnp.zeros((1,), padded.dtype), jnp.cumsum(padded)[:-1]])
    ends = starts + padded
    csum = jnp.cumsum(onehot, axis=0) - onehot
    rank = jnp.sum(csum * onehot, axis=1)
    start_a = jnp.sum(starts[None, :] * onehot, axis=1)
    pos2 = (start_a + rank).astype(jnp.int32).reshape(_T, _K)
    beid = jnp.minimum(
        jnp.sum((jnp.arange(_NB)[:, None] * _B >= ends[None, :])
                .astype(jnp.int32), axis=1),
        _E - 1).astype(jnp.int32)
    return pos2[:, 0], pos2[:, 1], beid


@functools.partial(
    pl.kernel, mesh=_mesh,
    out_type=jax.ShapeDtypeStruct((_P, _D), jnp.float32),
    scratch_types=[
        pltpu.VMEM((_CH, _D), jnp.float32),
        pltpu.VMEM((_CH,), jnp.int32),
        pltpu.VMEM((_CH,), jnp.int32),
        pltpu.SemaphoreType.DMA,
    ],
)
def _sc_dispatch(x_hbm, p0_hbm, p1_hbm, xs_hbm, rows_v, i0_v, i1_v, sem):
    wid = lax.axis_index("s") * 2 + lax.axis_index("c")
    base = wid * _TPW
    for c in range(_NCH):
        off = base + c * _CH
        pltpu.sync_copy(p0_hbm.at[pl.ds(off, _CH)], i0_v)
        pltpu.sync_copy(p1_hbm.at[pl.ds(off, _CH)], i1_v)
        pltpu.sync_copy(x_hbm.at[pl.ds(off, _CH)], rows_v)
        cp0 = pltpu.async_copy(rows_v, xs_hbm.at[i0_v], sem)
        cp1 = pltpu.async_copy(rows_v, xs_hbm.at[i1_v], sem)
        cp0.wait()
        cp1.wait()


def _gmm_body(eid_ref, xs_ref, we_ref, be_ref, ys_ref):
    ys_ref[...] = jnp.dot(xs_ref[...], we_ref[0],
                          preferred_element_type=jnp.float32) + be_ref[0]


def _grouped_matmul(block_eid, xs, we, be):
    grid_spec = pltpu.PrefetchScalarGridSpec(
        num_scalar_prefetch=1,
        grid=(_NB,),
        in_specs=[
            pl.BlockSpec((_B, _D), lambda b, eid: (b, 0)),
            pl.BlockSpec((1, _D, _D), lambda b, eid: (eid[b], 0, 0)),
            pl.BlockSpec((1, 1, _D), lambda b, eid: (eid[b], 0, 0)),
        ],
        out_specs=pl.BlockSpec((_B, _D), lambda b, eid: (b, 0)),
    )
    return pl.pallas_call(
        _gmm_body,
        grid_spec=grid_spec,
        out_shape=jax.ShapeDtypeStruct((_P, _D), jnp.float32),
    )(block_eid, xs, we, be)


@functools.partial(
    pl.kernel, mesh=_mesh,
    out_type=(jax.ShapeDtypeStruct((_T, _D), jnp.float32),
              jax.ShapeDtypeStruct((_T, _D), jnp.float32)),
    scratch_types=[
        pltpu.VMEM((_CH, _D), jnp.float32),
        pltpu.VMEM((_CH, _D), jnp.float32),
        pltpu.VMEM((_CH,), jnp.int32),
        pltpu.VMEM((_CH,), jnp.int32),
        pltpu.SemaphoreType.DMA,
    ],
)
def _sc_collect(ys_hbm, p0_hbm, p1_hbm, a_hbm, b_hbm,
                ra_v, rb_v, i0_v, i1_v, sem):
    wid = lax.axis_index("s") * 2 + lax.axis_index("c")
    base = wid * _TPW
    for c in range(_NCH):
        off = base + c * _CH
        pltpu.sync_copy(p0_hbm.at[pl.ds(off, _CH)], i0_v)
        pltpu.sync_copy(p1_hbm.at[pl.ds(off, _CH)], i1_v)
        ca = pltpu.async_copy(ys_hbm.at[i0_v], ra_v, sem)
        cb = pltpu.async_copy(ys_hbm.at[i1_v], rb_v, sem)
        ca.wait()
        cb.wait()
        pltpu.sync_copy(ra_v, a_hbm.at[pl.ds(off, _CH)])
        pltpu.sync_copy(rb_v, b_hbm.at[pl.ds(off, _CH)])


def _combine_body(a_ref, b_ref, p_ref, out_ref):
    out_ref[...] = (p_ref[:, 0:1] * a_ref[...] + p_ref[:, 1:2] * b_ref[...])


def _combine(a, b, probs):
    return pl.pallas_call(
        _combine_body,
        grid=(_T // _TT,),
        in_specs=[
            pl.BlockSpec((_TT, _D), lambda t: (t, 0)),
            pl.BlockSpec((_TT, _D), lambda t: (t, 0)),
            pl.BlockSpec((_TT, _K), lambda t: (t, 0)),
        ],
        out_specs=pl.BlockSpec((_TT, _D), lambda t: (t, 0)),
        out_shape=jax.ShapeDtypeStruct((_T, _D), jnp.float32),
    )(a, b, probs)


def kernel(inputs, Wg, bg, We, be):
    idx, probs = _router(inputs, Wg, bg.reshape(1, _E))
    pos0, pos1, block_eid = _dispatch_plan(idx)
    xs = _sc_dispatch(inputs, pos0, pos1)
    ys = _grouped_matmul(block_eid, xs, We, be.reshape(_E, 1, _D))
    a, b = _sc_collect(ys, pos0, pos1)
    out = _combine(a, b, probs)
    return (out, probs)
